# initial kernel scaffold (unmeasured)
import jax
import jax.numpy as jnp
from jax import lax
from jax.experimental import pallas as pl
from jax.experimental.pallas import tpu as pltpu

B, S, H, Dh, Dr = 2, 512, 16, 128, 32
D = 2048
BS = B * S
SCALE = (Dh + Dr) ** -0.5
BF = jnp.bfloat16


def kernel(x, Wdkv, Wuk, Wuv, Wq, Wqr, Wkr, Wo):
    x2d = x.reshape(BS, D).astype(BF)
    ws = [w.astype(BF) for w in (Wdkv, Wuk, Wuv, Wq, Wqr, Wkr, Wo)]

    def body(x_ref, Wdkv_ref, Wuk_ref, Wuv_ref, Wq_ref, Wqr_ref, Wkr_ref,
             Wo_ref, out_ref, kvs_ref, kvr_ref, o_ref, send_sem, recv_sem):
        my_x = lax.axis_index("x")
        my_y = lax.axis_index("y")
        nbr = (my_x, 1 - my_y)

        xv = x_ref[...]
        c = jnp.dot(xv, Wdkv_ref[...],
                    preferred_element_type=jnp.float32).astype(BF)
        kvs_ref[0] = jnp.dot(c, Wuk_ref[...],
                             preferred_element_type=jnp.float32).astype(BF)
        kvs_ref[1] = jnp.dot(c, Wuv_ref[...],
                             preferred_element_type=jnp.float32).astype(BF)

        barrier_sem = pltpu.get_barrier_semaphore()
        pl.semaphore_signal(barrier_sem, inc=1, device_id=nbr,
                            device_id_type=pl.DeviceIdType.MESH)
        pl.semaphore_wait(barrier_sem, 1)

        rdma = pltpu.make_async_remote_copy(
            src_ref=kvs_ref, dst_ref=kvr_ref,
            send_sem=send_sem, recv_sem=recv_sem,
            device_id=nbr, device_id_type=pl.DeviceIdType.MESH,
        )
        rdma.start()

        Q = jnp.dot(xv, Wq_ref[...],
                    preferred_element_type=jnp.float32).astype(BF)
        Qr = jnp.dot(xv, Wqr_ref[...],
                     preferred_element_type=jnp.float32).astype(BF)
        Kr = jnp.dot(xv, Wkr_ref[...],
                     preferred_element_type=jnp.float32).astype(BF)

        rdma.wait()
        K = kvs_ref[0] + kvr_ref[0]
        V = kvs_ref[1] + kvr_ref[1]

        for b in range(B):
            r0 = b * S
            Krb = Kr[r0:r0 + S, :]
            for h in range(H):
                c0 = h * Dh
                Qbh = Q[r0:r0 + S, c0:c0 + Dh]
                Kbh = K[r0:r0 + S, c0:c0 + Dh]
                Vbh = V[r0:r0 + S, c0:c0 + Dh]
                Qrbh = Qr[r0:r0 + S, h * Dr:(h + 1) * Dr]
                s = lax.dot_general(Qbh, Kbh, (((1,), (1,)), ((), ())),
                                    preferred_element_type=jnp.float32)
                s = s + lax.dot_general(Qrbh, Krb, (((1,), (1,)), ((), ())),
                                        preferred_element_type=jnp.float32)
                s = s * SCALE
                m = jnp.max(s, axis=-1, keepdims=True)
                p = jnp.exp(s - m)
                p = p / jnp.sum(p, axis=-1, keepdims=True)
                o_ref[r0:r0 + S, c0:c0 + Dh] = jnp.dot(
                    p.astype(BF), Vbh,
                    preferred_element_type=jnp.float32).astype(BF)

        out = jnp.dot(o_ref[...], Wo_ref[...],
                      preferred_element_type=jnp.float32)
        out_ref[...] = out.reshape(B, S, D)

    return pl.pallas_call(
        body,
        out_shape=jax.ShapeDtypeStruct((B, S, D), jnp.float32),
        in_specs=[pl.BlockSpec(memory_space=pltpu.VMEM)] * 8,
        out_specs=pl.BlockSpec(memory_space=pltpu.VMEM),
        scratch_shapes=[
            pltpu.VMEM((2, BS, D), BF),
            pltpu.VMEM((2, BS, D), BF),
            pltpu.VMEM((BS, D), BF),
            pltpu.SemaphoreType.DMA,
            pltpu.SemaphoreType.DMA,
        ],
        compiler_params=pltpu.CompilerParams(collective_id=0),
    )(x2d, *ws)


# baseline (device time: 182757 ns/iter reference)
import jax
import jax.numpy as jnp
from jax import lax
from jax.experimental import pallas as pl
from jax.experimental.pallas import tpu as pltpu

B, S, H, Dh, Dr = 2, 512, 16, 128, 32
D = 2048
BS = B * S
SCALE = (Dh + Dr) ** -0.5
BF = jnp.bfloat16


def kernel(x, Wdkv, Wuk, Wuv, Wq, Wqr, Wkr, Wo):
    x2d = x.reshape(BS, D).astype(BF)
    ws = [w.astype(BF) for w in (Wdkv, Wuk, Wuv, Wq, Wqr, Wkr, Wo)]

    def body(x_ref, Wdkv_ref, Wuk_ref, Wuv_ref, Wq_ref, Wqr_ref, Wkr_ref,
             Wo_ref, out_ref, kvs_ref, kvr_ref, o_ref, q_ref, qr_ref, kr_ref,
             send_sem, recv_sem):
        my_x = lax.axis_index("x")
        my_y = lax.axis_index("y")
        nbr = (my_x, 1 - my_y)

        xv = x_ref[...]
        c = jnp.dot(xv, Wdkv_ref[...],
                    preferred_element_type=jnp.float32).astype(BF)
        kvs_ref[0] = jnp.dot(c, Wuk_ref[...],
                             preferred_element_type=jnp.float32).astype(BF)
        kvs_ref[1] = jnp.dot(c, Wuv_ref[...],
                             preferred_element_type=jnp.float32).astype(BF)

        barrier_sem = pltpu.get_barrier_semaphore()
        pl.semaphore_signal(barrier_sem, inc=1, device_id=nbr,
                            device_id_type=pl.DeviceIdType.MESH)
        pl.semaphore_wait(barrier_sem, 1)

        rdma = pltpu.make_async_remote_copy(
            src_ref=kvs_ref, dst_ref=kvr_ref,
            send_sem=send_sem, recv_sem=recv_sem,
            device_id=nbr, device_id_type=pl.DeviceIdType.MESH,
        )
        rdma.start()

        q_ref[...] = jnp.dot(xv, Wq_ref[...],
                             preferred_element_type=jnp.float32).astype(BF)
        qr_ref[...] = jnp.dot(xv, Wqr_ref[...],
                              preferred_element_type=jnp.float32).astype(BF)
        kr_ref[...] = jnp.dot(xv, Wkr_ref[...],
                              preferred_element_type=jnp.float32).astype(BF)

        rdma.wait()
        kvs_ref[0] = kvs_ref[0] + kvr_ref[0]
        kvs_ref[1] = kvs_ref[1] + kvr_ref[1]

        for b in range(B):
            r0 = b * S
            Krb = kr_ref[r0:r0 + S, :]
            for h in range(H):
                c0 = h * Dh
                Qbh = q_ref[r0:r0 + S, c0:c0 + Dh]
                Kbh = kvs_ref[0, r0:r0 + S, c0:c0 + Dh]
                Vbh = kvs_ref[1, r0:r0 + S, c0:c0 + Dh]
                Qrbh = qr_ref[r0:r0 + S, h * Dr:(h + 1) * Dr]
                s = lax.dot_general(Qbh, Kbh, (((1,), (1,)), ((), ())),
                                    preferred_element_type=jnp.float32)
                s = s + lax.dot_general(Qrbh, Krb, (((1,), (1,)), ((), ())),
                                        preferred_element_type=jnp.float32)
                s = s * SCALE
                m = jnp.max(s, axis=-1, keepdims=True)
                p = jnp.exp(s - m)
                p = p / jnp.sum(p, axis=-1, keepdims=True)
                o_ref[r0:r0 + S, c0:c0 + Dh] = jnp.dot(
                    p.astype(BF), Vbh,
                    preferred_element_type=jnp.float32).astype(BF)

        for b in range(B):
            out_ref[b] = jnp.dot(o_ref[b * S:(b + 1) * S, :], Wo_ref[...],
                                 preferred_element_type=jnp.float32)

    return pl.pallas_call(
        body,
        out_shape=jax.ShapeDtypeStruct((B, S, D), jnp.float32),
        in_specs=[pl.BlockSpec(memory_space=pltpu.VMEM)] * 8,
        out_specs=pl.BlockSpec(memory_space=pltpu.VMEM),
        scratch_shapes=[
            pltpu.VMEM((2, BS, D), BF),
            pltpu.VMEM((2, BS, D), BF),
            pltpu.VMEM((BS, D), BF),
            pltpu.VMEM((BS, D), BF),
            pltpu.VMEM((BS, H * Dr), BF),
            pltpu.VMEM((BS, Dr), BF),
            pltpu.SemaphoreType.DMA,
            pltpu.SemaphoreType.DMA,
        ],
        compiler_params=pltpu.CompilerParams(
            collective_id=0, vmem_limit_bytes=100 * 1024 * 1024),
    )(x2d, *ws)


# device time: 130986 ns/iter; 1.3952x vs baseline; 1.3952x over previous
import jax
import jax.numpy as jnp
from jax import lax
from jax.experimental import pallas as pl
from jax.experimental.pallas import tpu as pltpu

B, S, H, Dh, Dr = 2, 512, 16, 128, 32
D = 2048
BS = B * S
HL = H // 2
DQ = HL * Dh
SCALE = (Dh + Dr) ** -0.5
BF = jnp.bfloat16
F32 = jnp.float32
MESH = pl.DeviceIdType.MESH


def kernel(x, Wdkv, Wuk, Wuv, Wq, Wqr, Wkr, Wo):
    x2d = x.reshape(BS, D).astype(BF)
    ws = [w.astype(BF) for w in (Wdkv, Wuk, Wuv, Wq, Wqr, Wkr, Wo)]

    def body(x_ref, Wdkv_ref, Wuk_ref, Wuv_ref, Wq_ref, Wqr_ref, Wkr_ref,
             Wo_ref, out_ref,
             wdkv_r, wuk_r, wuv_r,
             kq_ref, vq_ref, oq_ref,
             ops_ref, opr_ref,
             ogs_ref, ogr_ref,
             wsend, wrecv, psend, precv, gsend, grecv):
        my_x = lax.axis_index("x")
        my_y = lax.axis_index("y")
        ynbr = (my_x, 1 - my_y)
        xnbr = (1 - my_x, my_y)

        barrier_sem = pltpu.get_barrier_semaphore()
        pl.semaphore_signal(barrier_sem, inc=1, device_id=ynbr,
                            device_id_type=MESH)
        pl.semaphore_signal(barrier_sem, inc=1, device_id=xnbr,
                            device_id_type=MESH)
        pl.semaphore_wait(barrier_sem, 2)

        w_rdmas = []
        for i, (src, dst) in enumerate(
                ((Wdkv_ref, wdkv_r), (Wuk_ref, wuk_r), (Wuv_ref, wuv_r))):
            r = pltpu.make_async_remote_copy(
                src_ref=src, dst_ref=dst,
                send_sem=wsend.at[i], recv_sem=wrecv.at[i],
                device_id=ynbr, device_id_type=MESH)
            r.start()
            w_rdmas.append(r)

        xb = x_ref[pl.ds(my_x * S, S), :]
        c_my = jnp.dot(xb, Wdkv_ref[...],
                       preferred_element_type=F32).astype(BF)
        q = jnp.dot(xb, Wq_ref[:, pl.ds(my_y * DQ, DQ)],
                    preferred_element_type=F32).astype(BF)
        qr = jnp.dot(xb, Wqr_ref[:, pl.ds(my_y * HL * Dr, HL * Dr)],
                     preferred_element_type=F32).astype(BF)
        kr = jnp.dot(xb, Wkr_ref[...],
                     preferred_element_type=F32).astype(BF)

        for r in w_rdmas:
            r.wait()

        c_nb = jnp.dot(xb, wdkv_r[...],
                       preferred_element_type=F32).astype(BF)
        myc = pl.ds(my_y * DQ, DQ)
        kq_ref[...] = (
            jnp.dot(c_my, Wuk_ref[:, myc], preferred_element_type=F32)
            + jnp.dot(c_nb, wuk_r[:, myc], preferred_element_type=F32)
        ).astype(BF)
        vq_ref[...] = (
            jnp.dot(c_my, Wuv_ref[:, myc], preferred_element_type=F32)
            + jnp.dot(c_nb, wuv_r[:, myc], preferred_element_type=F32)
        ).astype(BF)

        for h in range(HL):
            c0 = h * Dh
            s = lax.dot_general(
                q[:, c0:c0 + Dh], kq_ref[:, c0:c0 + Dh],
                (((1,), (1,)), ((), ())), preferred_element_type=F32)
            s = s + lax.dot_general(
                qr[:, h * Dr:(h + 1) * Dr], kr,
                (((1,), (1,)), ((), ())), preferred_element_type=F32)
            s = s * SCALE
            m = jnp.max(s, axis=-1, keepdims=True)
            p = jnp.exp(s - m)
            p = p / jnp.sum(p, axis=-1, keepdims=True)
            oq_ref[:, c0:c0 + Dh] = jnp.dot(
                p.astype(BF), vq_ref[:, c0:c0 + Dh],
                preferred_element_type=F32).astype(BF)

        op = jnp.dot(oq_ref[...], Wo_ref[pl.ds(my_y * DQ, DQ), :],
                     preferred_element_type=F32)
        ops_ref[...] = op.astype(BF)
        pr = pltpu.make_async_remote_copy(
            src_ref=ops_ref, dst_ref=opr_ref,
            send_sem=psend, recv_sem=precv,
            device_id=ynbr, device_id_type=MESH)
        pr.start()
        pr.wait()
        out_b = op + opr_ref[...].astype(F32)

        ogs_ref[...] = out_b.astype(BF)
        gr = pltpu.make_async_remote_copy(
            src_ref=ogs_ref, dst_ref=ogr_ref,
            send_sem=gsend, recv_sem=grecv,
            device_id=xnbr, device_id_type=MESH)
        gr.start()
        out_ref[my_x, :, :] = out_b
        gr.wait()
        out_ref[1 - my_x, :, :] = ogr_ref[...].astype(F32)

    return pl.pallas_call(
        body,
        out_shape=jax.ShapeDtypeStruct((B, S, D), F32),
        in_specs=[pl.BlockSpec(memory_space=pltpu.VMEM)] * 8,
        out_specs=pl.BlockSpec(memory_space=pltpu.VMEM),
        scratch_shapes=[
            pltpu.VMEM((D, 128), BF),
            pltpu.VMEM((128, D), BF),
            pltpu.VMEM((128, D), BF),
            pltpu.VMEM((S, DQ), BF),
            pltpu.VMEM((S, DQ), BF),
            pltpu.VMEM((S, DQ), BF),
            pltpu.VMEM((S, D), BF),
            pltpu.VMEM((S, D), BF),
            pltpu.VMEM((S, D), BF),
            pltpu.VMEM((S, D), BF),
            pltpu.SemaphoreType.DMA((3,)),
            pltpu.SemaphoreType.DMA((3,)),
            pltpu.SemaphoreType.DMA,
            pltpu.SemaphoreType.DMA,
            pltpu.SemaphoreType.DMA,
            pltpu.SemaphoreType.DMA,
        ],
        compiler_params=pltpu.CompilerParams(
            collective_id=0, vmem_limit_bytes=100 * 1024 * 1024),
    )(x2d, *ws)


# device time: 94744 ns/iter; 1.9290x vs baseline; 1.3825x over previous
import jax
import jax.numpy as jnp
from jax import lax
from jax.experimental import pallas as pl
from jax.experimental.pallas import tpu as pltpu

B, S, H, Dh, Dr = 2, 512, 16, 128, 32
D = 2048
BS = B * S
HL = H // 2
DQ = HL * Dh
DRQ = HL * Dr
SCALE = (Dh + Dr) ** -0.5
BF = jnp.bfloat16
F32 = jnp.float32
MESH = pl.DeviceIdType.MESH


def kernel(x, Wdkv, Wuk, Wuv, Wq, Wqr, Wkr, Wo):
    x2d = x.reshape(BS, D)

    def body(x_hbm, Wdkv_ref, Wuk_ref, Wuv_ref, Wq_hbm, Wqr_hbm, Wkr_ref,
             Wo_hbm, out_ref,
             xb32, wq32, wqr32, wo32,
             wdkv_s, wuk_s, wuv_s,
             wdkv_r, wuk_r, wuv_r,
             kq_ref, vq_ref, oq_ref,
             ops_ref, opr_ref, ogr_ref,
             ldma, wsend, wrecv, psend, precv, gsend, grecv):
        my_x = lax.axis_index("x")
        my_y = lax.axis_index("y")
        ynbr = (my_x, 1 - my_y)
        xnbr = (1 - my_x, my_y)

        cp_x = pltpu.make_async_copy(
            x_hbm.at[pl.ds(my_x * S, S), :], xb32, ldma.at[0])
        cp_wq = pltpu.make_async_copy(
            Wq_hbm.at[:, pl.ds(my_y * DQ, DQ)], wq32, ldma.at[1])
        cp_wqr = pltpu.make_async_copy(
            Wqr_hbm.at[:, pl.ds(my_y * DRQ, DRQ)], wqr32, ldma.at[2])
        cp_wo = pltpu.make_async_copy(
            Wo_hbm.at[pl.ds(my_y * DQ, DQ), :], wo32, ldma.at[3])
        cp_x.start()
        cp_wq.start()
        cp_wqr.start()
        cp_wo.start()

        wdkv_s[...] = Wdkv_ref[...].astype(BF)
        wuk_s[...] = Wuk_ref[...].astype(BF)
        wuv_s[...] = Wuv_ref[...].astype(BF)

        barrier_sem = pltpu.get_barrier_semaphore()
        pl.semaphore_signal(barrier_sem, inc=1, device_id=ynbr,
                            device_id_type=MESH)
        pl.semaphore_signal(barrier_sem, inc=1, device_id=xnbr,
                            device_id_type=MESH)
        pl.semaphore_wait(barrier_sem, 2)

        w_rdmas = []
        for i, (src, dst) in enumerate(
                ((wdkv_s, wdkv_r), (wuk_s, wuk_r), (wuv_s, wuv_r))):
            r = pltpu.make_async_remote_copy(
                src_ref=src, dst_ref=dst,
                send_sem=wsend.at[i], recv_sem=wrecv.at[i],
                device_id=ynbr, device_id_type=MESH)
            r.start()
            w_rdmas.append(r)

        cp_x.wait()
        xb = xb32[...].astype(BF)
        c_my = jnp.dot(xb, wdkv_s[...],
                       preferred_element_type=F32).astype(BF)
        cp_wq.wait()
        q = (jnp.dot(xb, wq32[...].astype(BF),
                     preferred_element_type=F32) * SCALE).astype(BF)
        cp_wqr.wait()
        qr = (jnp.dot(xb, wqr32[...].astype(BF),
                      preferred_element_type=F32) * SCALE).astype(BF)
        kr = jnp.dot(xb, Wkr_ref[...].astype(BF),
                     preferred_element_type=F32).astype(BF)

        for r in w_rdmas:
            r.wait()

        c_nb = jnp.dot(xb, wdkv_r[...],
                       preferred_element_type=F32).astype(BF)
        myc = pl.ds(my_y * DQ, DQ)
        kq_ref[...] = (
            jnp.dot(c_my, wuk_s[:, myc], preferred_element_type=F32)
            + jnp.dot(c_nb, wuk_r[:, myc], preferred_element_type=F32)
        ).astype(BF)
        vq_ref[...] = (
            jnp.dot(c_my, wuv_s[:, myc], preferred_element_type=F32)
            + jnp.dot(c_nb, wuv_r[:, myc], preferred_element_type=F32)
        ).astype(BF)

        for h in range(HL):
            c0 = h * Dh
            s = lax.dot_general(
                q[:, c0:c0 + Dh], kq_ref[:, c0:c0 + Dh],
                (((1,), (1,)), ((), ())), preferred_element_type=F32)
            s = s + lax.dot_general(
                qr[:, h * Dr:(h + 1) * Dr], kr,
                (((1,), (1,)), ((), ())), preferred_element_type=F32)
            m = jnp.max(s, axis=-1, keepdims=True)
            p = jnp.exp(s - m)
            o = jnp.dot(p.astype(BF), vq_ref[:, c0:c0 + Dh],
                        preferred_element_type=F32)
            oq_ref[:, c0:c0 + Dh] = (
                o / jnp.sum(p, axis=-1, keepdims=True)).astype(BF)

        cp_wo.wait()
        op = jnp.dot(oq_ref[...], wo32[...].astype(BF),
                     preferred_element_type=F32)
        ops_ref[...] = op.astype(BF)
        pr = pltpu.make_async_remote_copy(
            src_ref=ops_ref, dst_ref=opr_ref,
            send_sem=psend, recv_sem=precv,
            device_id=ynbr, device_id_type=MESH)
        pr.start()
        pr.wait()
        out_b = op + opr_ref[...].astype(F32)

        ops_ref[...] = out_b.astype(BF)
        gr = pltpu.make_async_remote_copy(
            src_ref=ops_ref, dst_ref=ogr_ref,
            send_sem=gsend, recv_sem=grecv,
            device_id=xnbr, device_id_type=MESH)
        gr.start()
        out_ref[my_x, :, :] = out_b
        gr.wait()
        out_ref[1 - my_x, :, :] = ogr_ref[...].astype(F32)

    vmem = pl.BlockSpec(memory_space=pltpu.VMEM)
    hbm = pl.BlockSpec(memory_space=pl.ANY)
    return pl.pallas_call(
        body,
        out_shape=jax.ShapeDtypeStruct((B, S, D), F32),
        in_specs=[hbm, vmem, vmem, vmem, hbm, hbm, vmem, hbm],
        out_specs=vmem,
        scratch_shapes=[
            pltpu.VMEM((S, D), F32),
            pltpu.VMEM((D, DQ), F32),
            pltpu.VMEM((D, DRQ), F32),
            pltpu.VMEM((DQ, D), F32),
            pltpu.VMEM((D, 128), BF),
            pltpu.VMEM((128, D), BF),
            pltpu.VMEM((128, D), BF),
            pltpu.VMEM((D, 128), BF),
            pltpu.VMEM((128, D), BF),
            pltpu.VMEM((128, D), BF),
            pltpu.VMEM((S, DQ), BF),
            pltpu.VMEM((S, DQ), BF),
            pltpu.VMEM((S, DQ), BF),
            pltpu.VMEM((S, D), BF),
            pltpu.VMEM((S, D), BF),
            pltpu.VMEM((S, D), BF),
            pltpu.SemaphoreType.DMA((4,)),
            pltpu.SemaphoreType.DMA((3,)),
            pltpu.SemaphoreType.DMA((3,)),
            pltpu.SemaphoreType.DMA,
            pltpu.SemaphoreType.DMA,
            pltpu.SemaphoreType.DMA,
            pltpu.SemaphoreType.DMA,
        ],
        compiler_params=pltpu.CompilerParams(
            collective_id=0, vmem_limit_bytes=100 * 1024 * 1024),
    )(x2d, Wdkv, Wuk, Wuv, Wq, Wqr, Wkr, Wo)
